# baseline (speedup 1.0000x reference)
import dataclasses
import functools

import jax
import jax.numpy as jnp
import numpy as np
from jax import lax
from jax.experimental import pallas as pl
from jax.experimental.pallas import tpu as pltpu
from jax.experimental.pallas import tpu_sc as plsc

C = 128
CAP = 512
D = 512
B = 8192
BLK = 2048
NSTEP = B // BLK
NZSTEP = (C * CAP) // BLK

NW = 32
ROWS_PER_TILE = (C * CAP) // NW
NCLS_TILE = ROWS_PER_TILE // CAP
G = 64

_U = np.triu(np.ones((BLK, BLK), np.float32)).astype(jnp.bfloat16)


def _prep_body(u_ref, label_ref, d_ref, len_ref, carry_ref):
    i = pl.program_id(0)

    @pl.when(i == 0)
    def _():
        carry_ref[...] = jnp.zeros_like(carry_ref)

    @pl.when(i < NSTEP)
    def _():
        lb = label_ref[0, 0, :]
        cls = lax.broadcasted_iota(jnp.int32, (C, BLK), 0)
        onehot = cls == lb[None, :]
        csum = lax.dot_general(
            onehot.astype(jnp.bfloat16), u_ref[...],
            dimension_numbers=(((1,), (0,)), ((), ())),
            preferred_element_type=jnp.float32)
        total = csum + carry_ref[...]
        rank = jnp.sum(jnp.where(onehot, total, 0.0), axis=0) - 1.0
        rank_i = rank.astype(jnp.int32)
        dd = lb * CAP + rank_i
        dd = jnp.where(rank_i < CAP, dd, jnp.int32(2**30))
        d_ref[0, 0, :] = dd
        carry_ref[...] = carry_ref[...] + csum[:, BLK - 1:BLK]

    @pl.when(i == NSTEP - 1)
    def _():
        len_ref[...] = carry_ref[...].astype(jnp.int32)


_prep = pl.pallas_call(
    _prep_body,
    grid=(NSTEP,),
    in_specs=[
        pl.BlockSpec((BLK, BLK), lambda i: (0, 0)),
        pl.BlockSpec((1, 1, BLK), lambda i: (i, 0, 0)),
    ],
    out_specs=[
        pl.BlockSpec((1, 1, BLK), lambda i: (i, 0, 0)),
        pl.BlockSpec((C, 1), lambda i: (0, 0)),
    ],
    out_shape=[
        jax.ShapeDtypeStruct((NSTEP, 1, BLK), jnp.int32),
        jax.ShapeDtypeStruct((C, 1), jnp.int32),
    ],
    scratch_shapes=[pltpu.VMEM((C, 1), jnp.float32)],
)

ZSPLIT = 2 * G


def _zeros_body(mem_ref):
    mem_ref[...] = jnp.zeros_like(mem_ref)


_zeros = pl.pallas_call(
    _zeros_body,
    grid=(C, (CAP - ZSPLIT) // ZSPLIT),
    out_specs=pl.BlockSpec((1, ZSPLIT, D), lambda c, j: (c, j + 1, 0)),
    out_shape=jax.ShapeDtypeStruct((C, CAP, D), jnp.float32),
)


def _sc_write_body(f_hbm, d_hbm, mem_hbm, d_v, src_v, gidx_v, buf0, buf1,
                   zbuf, g0, g1, w0, w1):
    wid = lax.axis_index("s") * 2 + lax.axis_index("c")
    base = wid * ROWS_PER_TILE
    pltpu.sync_copy(mem_hbm.at[pl.ds(base + CAP - G, G)], zbuf)
    pltpu.sync_copy(d_hbm, d_v)

    @pl.loop(0, ROWS_PER_TILE, step=16, unroll=8)
    def _(i):
        src_v[pl.ds(i, 16)] = jnp.full((16,), B, jnp.int32)

    @pl.loop(0, B, step=16, unroll=4)
    def _(i):
        vd = d_v[pl.ds(i, 16)]
        loc = vd - base
        m = (loc >= 0) & (loc < ROWS_PER_TILE)
        locc = jnp.clip(loc, 0, ROWS_PER_TILE - 1)
        vi = lax.iota(jnp.int32, 16) + i
        plsc.store_scatter(src_v, [locc], vi, mask=m)

    for cls in range(NCLS_TILE):
        cbase = cls * CAP

        def _cnt(k, acc, cbase=cbase):
            return acc + jnp.sum(
                (src_v[pl.ds(cbase + k * 16, 16)] < B).astype(jnp.int32))

        m = lax.fori_loop(0, CAP // 16, _cnt, jnp.int32(0))
        t = (m + G - 1) // G

        def _pair(j, _, cbase=cbase, t=t):
            r0 = cbase + (2 * j) * G
            r1 = r0 + G
            more = 2 * j + 1 < t - 1
            pltpu.async_copy(f_hbm.at[src_v.at[pl.ds(r0, G)]], buf0, g0)

            @pl.when(more)
            def _():
                pltpu.async_copy(f_hbm.at[src_v.at[pl.ds(r1, G)]], buf1, g1)

            pltpu.make_async_copy(
                f_hbm.at[src_v.at[pl.ds(r0, G)]], buf0, g0).wait()
            pltpu.async_copy(buf0, mem_hbm.at[pl.ds(base + r0, G)], w0)

            @pl.when(more)
            def _():
                pltpu.make_async_copy(
                    f_hbm.at[src_v.at[pl.ds(r1, G)]], buf1, g1).wait()
                pltpu.async_copy(buf1, mem_hbm.at[pl.ds(base + r1, G)], w1)

            pltpu.make_async_copy(
                buf0, mem_hbm.at[pl.ds(base + r0, G)], w0).wait()

            @pl.when(more)
            def _():
                pltpu.make_async_copy(
                    buf1, mem_hbm.at[pl.ds(base + r1, G)], w1).wait()

            return 0

        npair = (jnp.maximum(t - 1, 0) + 1) // 2
        lax.fori_loop(0, npair, _pair, 0)

        @pl.when(t > 0)
        def _(cbase=cbase, m=m, t=t):
            rb = cbase + (t - 1) * G

            @pl.loop(0, G, step=16)
            def _(i2):
                v = src_v[pl.ds(rb + i2, 16)]
                spread = (base + rb + i2 + lax.iota(jnp.int32, 16)) & (B - 1)
                gidx_v[pl.ds(i2, 16)] = jnp.where(v < B, v, spread)

            pltpu.async_copy(f_hbm.at[gidx_v], buf0, g0).wait()
            k = m - (t - 1) * G

            def _zrow(j, _):
                for l in range(D // 16):
                    buf0[j, pl.ds(l * 16, 16)] = jnp.zeros((16,), jnp.float32)
                return 0

            lax.fori_loop(k, G, _zrow, 0)
            pltpu.sync_copy(buf0, mem_hbm.at[pl.ds(base + rb, G)])

        @pl.when(t < 1)
        def _(cbase=cbase):
            pltpu.async_copy(zbuf, mem_hbm.at[pl.ds(base + cbase, G)], w0)

        @pl.when(t < 2)
        def _(cbase=cbase):
            pltpu.async_copy(zbuf, mem_hbm.at[pl.ds(base + cbase + G, G)], w1)

        @pl.when(t < 1)
        def _(cbase=cbase):
            pltpu.make_async_copy(
                zbuf, mem_hbm.at[pl.ds(base + cbase, G)], w0).wait()

        @pl.when(t < 2)
        def _(cbase=cbase):
            pltpu.make_async_copy(
                zbuf, mem_hbm.at[pl.ds(base + cbase + G, G)], w1).wait()


@functools.cache
def _sc_write():
    mesh = plsc.VectorSubcoreMesh(core_axis_name="c", subcore_axis_name="s")
    cp = pltpu.CompilerParams()
    if "needs_layout_passes" in pltpu.CompilerParams.__dataclass_fields__:
        cp = dataclasses.replace(cp, needs_layout_passes=False)
    return pl.kernel(
        _sc_write_body,
        out_type=(),
        mesh=mesh,
        compiler_params=cp,
        scratch_types=[
            pltpu.VMEM((B,), jnp.int32),
            pltpu.VMEM((ROWS_PER_TILE,), jnp.int32),
            pltpu.VMEM((G,), jnp.int32),
            pltpu.VMEM((G, D), jnp.float32),
            pltpu.VMEM((G, D), jnp.float32),
            pltpu.VMEM((G, D), jnp.float32),
            pltpu.SemaphoreType.DMA,
            pltpu.SemaphoreType.DMA,
            pltpu.SemaphoreType.DMA,
            pltpu.SemaphoreType.DMA,
        ],
    )


def kernel(feature, label):
    u = jnp.asarray(_U)
    d3, mlen = _prep(u, label.reshape(NSTEP, 1, BLK))
    mem0 = _zeros()
    mem_ref = jax.new_ref(mem0.reshape(C * CAP, D))
    _sc_write()(feature, d3.reshape(B), mem_ref)
    return mem_ref[...].reshape(C, CAP, D), mlen.reshape(C)


# speedup vs baseline: 2.0670x; 2.0670x over previous
import dataclasses
import functools

import jax
import jax.numpy as jnp
import numpy as np
from jax import lax
from jax.experimental import pallas as pl
from jax.experimental.pallas import tpu as pltpu
from jax.experimental.pallas import tpu_sc as plsc

C = 128
CAP = 512
D = 512
B = 8192
BLK = 2048
NSTEP = B // BLK
NZSTEP = (C * CAP) // BLK

NW = 32
ROWS_PER_TILE = (C * CAP) // NW
NCLS_TILE = ROWS_PER_TILE // CAP
G = 64

_U = np.triu(np.ones((BLK, BLK), np.float32)).astype(jnp.bfloat16)


def _prep_body(u_ref, label_ref, d_ref, len_ref, mem_ref, carry_ref):
    i = pl.program_id(0)

    mem_ref[...] = jnp.zeros_like(mem_ref)

    @pl.when(i == 0)
    def _():
        carry_ref[...] = jnp.zeros_like(carry_ref)

    @pl.when(i < NSTEP)
    def _():
        lb = label_ref[0, 0, :]
        cls = lax.broadcasted_iota(jnp.int32, (C, BLK), 0)
        onehot = cls == lb[None, :]
        csum = lax.dot_general(
            onehot.astype(jnp.bfloat16), u_ref[...],
            dimension_numbers=(((1,), (0,)), ((), ())),
            preferred_element_type=jnp.float32)
        total = csum + carry_ref[...]
        rank = jnp.sum(jnp.where(onehot, total, 0.0), axis=0) - 1.0
        rank_i = rank.astype(jnp.int32)
        dd = lb * CAP + rank_i
        dd = jnp.where(rank_i < CAP, dd, jnp.int32(2**30))
        d_ref[0, 0, :] = dd
        carry_ref[...] = carry_ref[...] + csum[:, BLK - 1:BLK]

    @pl.when(i == NSTEP - 1)
    def _():
        len_ref[...] = carry_ref[...].astype(jnp.int32)


_prep = pl.pallas_call(
    _prep_body,
    grid=(NZSTEP,),
    in_specs=[
        pl.BlockSpec((BLK, BLK), lambda i: (0, 0)),
        pl.BlockSpec((1, 1, BLK), lambda i: (jnp.minimum(i, NSTEP - 1), 0, 0)),
    ],
    out_specs=[
        pl.BlockSpec((1, 1, BLK), lambda i: (jnp.minimum(i, NSTEP - 1), 0, 0)),
        pl.BlockSpec((C, 1), lambda i: (0, 0)),
        pl.BlockSpec((BLK, D), lambda i: (i, 0)),
    ],
    out_shape=[
        jax.ShapeDtypeStruct((NSTEP, 1, BLK), jnp.int32),
        jax.ShapeDtypeStruct((C, 1), jnp.int32),
        jax.ShapeDtypeStruct((C * CAP, D), jnp.float32),
    ],
    scratch_shapes=[pltpu.VMEM((C, 1), jnp.float32)],
)


def _sc_write_body(f_hbm, d_hbm, mem_hbm, d_v, src_v, gidx_v, buf0, buf1,
                   g0, g1, w0, w1):
    wid = lax.axis_index("s") * 2 + lax.axis_index("c")
    base = wid * ROWS_PER_TILE
    pltpu.sync_copy(d_hbm, d_v)

    @pl.loop(0, ROWS_PER_TILE, step=16, unroll=8)
    def _(i):
        src_v[pl.ds(i, 16)] = jnp.full((16,), B, jnp.int32)

    @pl.loop(0, B, step=16, unroll=4)
    def _(i):
        vd = d_v[pl.ds(i, 16)]
        loc = vd - base
        m = (loc >= 0) & (loc < ROWS_PER_TILE)
        locc = jnp.clip(loc, 0, ROWS_PER_TILE - 1)
        vi = lax.iota(jnp.int32, 16) + i
        plsc.store_scatter(src_v, [locc], vi, mask=m)

    for cls in range(NCLS_TILE):
        cbase = cls * CAP

        def _cnt(k, acc, cbase=cbase):
            return acc + jnp.sum(
                (src_v[pl.ds(cbase + k * 16, 16)] < B).astype(jnp.int32))

        m = lax.fori_loop(0, CAP // 16, _cnt, jnp.int32(0))
        t = (m + G - 1) // G

        def _pair(j, _, cbase=cbase, t=t):
            r0 = cbase + (2 * j) * G
            r1 = r0 + G
            more = 2 * j + 1 < t - 1
            pltpu.async_copy(f_hbm.at[src_v.at[pl.ds(r0, G)]], buf0, g0)

            @pl.when(more)
            def _():
                pltpu.async_copy(f_hbm.at[src_v.at[pl.ds(r1, G)]], buf1, g1)

            pltpu.make_async_copy(
                f_hbm.at[src_v.at[pl.ds(r0, G)]], buf0, g0).wait()
            pltpu.async_copy(buf0, mem_hbm.at[pl.ds(base + r0, G)], w0)

            @pl.when(more)
            def _():
                pltpu.make_async_copy(
                    f_hbm.at[src_v.at[pl.ds(r1, G)]], buf1, g1).wait()
                pltpu.async_copy(buf1, mem_hbm.at[pl.ds(base + r1, G)], w1)

            pltpu.make_async_copy(
                buf0, mem_hbm.at[pl.ds(base + r0, G)], w0).wait()

            @pl.when(more)
            def _():
                pltpu.make_async_copy(
                    buf1, mem_hbm.at[pl.ds(base + r1, G)], w1).wait()

            return 0

        npair = (jnp.maximum(t - 1, 0) + 1) // 2
        lax.fori_loop(0, npair, _pair, 0)

        @pl.when(t > 0)
        def _(cbase=cbase, m=m, t=t):
            rb = cbase + (t - 1) * G

            @pl.loop(0, G, step=16)
            def _(i2):
                v = src_v[pl.ds(rb + i2, 16)]
                spread = (base + rb + i2 + lax.iota(jnp.int32, 16)) & (B - 1)
                gidx_v[pl.ds(i2, 16)] = jnp.where(v < B, v, spread)

            pltpu.async_copy(f_hbm.at[gidx_v], buf0, g0).wait()
            k = m - (t - 1) * G

            def _zrow(j, _):
                for l in range(D // 16):
                    buf0[j, pl.ds(l * 16, 16)] = jnp.zeros((16,), jnp.float32)
                return 0

            lax.fori_loop(k, G, _zrow, 0)
            pltpu.sync_copy(buf0, mem_hbm.at[pl.ds(base + rb, G)])


@functools.cache
def _sc_write():
    mesh = plsc.VectorSubcoreMesh(core_axis_name="c", subcore_axis_name="s")
    cp = pltpu.CompilerParams()
    if "needs_layout_passes" in pltpu.CompilerParams.__dataclass_fields__:
        cp = dataclasses.replace(cp, needs_layout_passes=False)
    return pl.kernel(
        _sc_write_body,
        out_type=(),
        mesh=mesh,
        compiler_params=cp,
        scratch_types=[
            pltpu.VMEM((B,), jnp.int32),
            pltpu.VMEM((ROWS_PER_TILE,), jnp.int32),
            pltpu.VMEM((G,), jnp.int32),
            pltpu.VMEM((G, D), jnp.float32),
            pltpu.VMEM((G, D), jnp.float32),
            pltpu.SemaphoreType.DMA,
            pltpu.SemaphoreType.DMA,
            pltpu.SemaphoreType.DMA,
            pltpu.SemaphoreType.DMA,
        ],
    )


def kernel(feature, label):
    u = jnp.asarray(_U)
    d3, mlen, mem0 = _prep(u, label.reshape(NSTEP, 1, BLK))
    mem_ref = jax.new_ref(mem0)
    _sc_write()(feature, d3.reshape(B), mem_ref)
    return mem_ref[...].reshape(C, CAP, D), mlen.reshape(C)
